# Initial kernel scaffold; baseline (speedup 1.0000x reference)
import jax
import jax.numpy as jnp
from jax.experimental import pallas as pl


def kernel(x_num, x_cat, E, W1, b1, W2, b2, W3, b3, WA, bA, WB, bB):
    raise NotImplementedError("write your pallas kernel here")


# speedup vs baseline: 2.1782x; 2.1782x over previous
import functools

import jax
import jax.numpy as jnp
from jax import lax
from jax.experimental import pallas as pl
from jax.experimental.pallas import tpu as pltpu
from jax.experimental.pallas import tpu_sc as plsc

_NC = 2
_NS = 16
_NW = _NC * _NS
_CH = 128
_NBUF = 8


def _sc_gather(e2, idx3, n_rows, d):
    nch = idx3.shape[1]
    r_per_w = nch * _CH
    ng = nch // _NBUF
    mesh = plsc.VectorSubcoreMesh(
        core_axis_name="c", subcore_axis_name="s",
        num_cores=_NC, num_subcores=_NS)

    def body(e_hbm, idx_hbm, out_hbm, idx_v, rows_v, *sems):
        wid = lax.axis_index("s") * _NC + lax.axis_index("c")
        base = wid * r_per_w
        pltpu.sync_copy(idx_hbm.at[wid], idx_v)
        for b in range(_NBUF):
            pltpu.async_copy(e_hbm.at[idx_v.at[b]], rows_v.at[b], sems[b])

        def round_body(g, carry):
            for b in range(_NBUF):
                j = g * _NBUF + b
                pltpu.make_async_copy(
                    e_hbm.at[idx_v.at[b]], rows_v.at[b], sems[b]).wait()
                pltpu.sync_copy(
                    rows_v.at[b], out_hbm.at[pl.ds(base + j * _CH, _CH)])

                @pl.when(g < ng - 1)
                def _():
                    pltpu.async_copy(
                        e_hbm.at[idx_v.at[j + _NBUF]], rows_v.at[b], sems[b])
            return carry

        lax.fori_loop(0, ng, round_body, 0)

    run = pl.kernel(
        body,
        out_type=jax.ShapeDtypeStruct((n_rows, d), jnp.float32),
        mesh=mesh,
        scratch_types=(
            [pltpu.VMEM((nch, _CH), jnp.int32),
             pltpu.VMEM((_NBUF, _CH, d), jnp.float32)]
            + [pltpu.SemaphoreType.DMA] * _NBUF),
        compiler_params=pltpu.CompilerParams(use_tc_tiling_on_sc=False),
    )
    return run(e2, idx3)


def _mlp(x_num, emb, w1n, w1e, b1, w2, b2, w3, b3, wab, bab):
    bsz, num_dim = x_num.shape
    ed = emb.shape[1]
    h1, h2, h3 = w1e.shape[1], w2.shape[1], w3.shape[1]
    blk = 1024

    def body(xn, xe, w1n_r, w1e_r, b1_r, w2_r, b2_r, w3_r, b3_r, wab_r,
             bab_r, out):
        a = jnp.dot(xn[...], w1n_r[...], preferred_element_type=jnp.float32)
        a += jnp.dot(xe[...], w1e_r[...], preferred_element_type=jnp.float32)
        h = jnp.maximum(a + b1_r[...], 0.0)
        h = jnp.maximum(
            jnp.dot(h, w2_r[...], preferred_element_type=jnp.float32)
            + b2_r[...], 0.0)
        h = jnp.maximum(
            jnp.dot(h, w3_r[...], preferred_element_type=jnp.float32)
            + b3_r[...], 0.0)
        out[...] = (jnp.dot(h, wab_r[...], preferred_element_type=jnp.float32)
                    + bab_r[...])

    const = lambda i: (0, 0)
    return pl.pallas_call(
        body,
        grid=(bsz // blk,),
        in_specs=[
            pl.BlockSpec((blk, num_dim), lambda i: (i, 0)),
            pl.BlockSpec((blk, ed), lambda i: (i, 0)),
            pl.BlockSpec((num_dim, h1), const),
            pl.BlockSpec((ed, h1), const),
            pl.BlockSpec((1, h1), const),
            pl.BlockSpec((h1, h2), const),
            pl.BlockSpec((1, h2), const),
            pl.BlockSpec((h2, h3), const),
            pl.BlockSpec((1, h3), const),
            pl.BlockSpec((h3, 2), const),
            pl.BlockSpec((1, 2), const),
        ],
        out_specs=pl.BlockSpec((blk, 2), lambda i: (i, 0)),
        out_shape=jax.ShapeDtypeStruct((bsz, 2), jnp.float32),
    )(x_num, emb, w1n, w1e, b1, w2, b2, w3, b3, wab, bab)


def kernel(x_num, x_cat, E, W1, b1, W2, b2, W3, b3, WA, bA, WB, bB):
    bsz, num_dim = x_num.shape
    f = x_cat.shape[1]
    v, d = E.shape[1], E.shape[2]
    n_rows = bsz * f
    nch = n_rows // (_NW * _CH)

    offs = (jnp.arange(f, dtype=jnp.int32) * v)[None, :]
    idx3 = (x_cat + offs).reshape(_NW, nch, _CH)
    e2 = E.reshape(f * v, d)

    emb = _sc_gather(e2, idx3, n_rows, d).reshape(bsz, f * d)

    wab = jnp.concatenate([WA, WB], axis=1)
    bab = jnp.concatenate([bA, bB])[None, :]
    out = _mlp(x_num, emb, W1[:num_dim], W1[num_dim:], b1[None, :],
               W2, b2[None, :], W3, b3[None, :], wab, bab)
    return out[:, 0], out[:, 1]
